# Initial kernel scaffold; baseline (speedup 1.0000x reference)
import jax
import jax.numpy as jnp
from jax.experimental import pallas as pl


def kernel(user_emb, item_emb, edge_index, adj_vals):
    raise NotImplementedError("write your pallas kernel here")


# speedup vs baseline: 5.4403x; 5.4403x over previous
import functools

import jax
import jax.numpy as jnp
from jax import lax
from jax.experimental import pallas as pl
from jax.experimental.pallas import tpu as pltpu
from jax.experimental.pallas import tpu_sc as plsc

N_USERS = 5000
N_ITEMS = 5000
N = N_USERS + N_ITEMS
E = 320000
D = 128
LEAKY = 0.5

NC = 2
NS = 16
NW = NC * NS
L = 16

NP = 10240
EPT = E // NW
K = 80
NSUB = EPT // K
RPT = NP // NS
RCHUNK = 128
NB = K // L


def _bcast_lane(v16, lane):
    idx = jnp.full((L,), lane, dtype=jnp.int32)
    return v16.at[idx].get(mode="promise_in_bounds")


_sc_mesh = plsc.VectorSubcoreMesh(core_axis_name="c", subcore_axis_name="s")


@functools.partial(
    pl.kernel,
    out_type=jax.ShapeDtypeStruct((NC, NP, D), jnp.float32),
    mesh=_sc_mesh,
    scratch_types=[
        pltpu.VMEM((EPT,), jnp.int32),
        pltpu.VMEM((EPT,), jnp.int32),
        pltpu.VMEM((EPT,), jnp.float32),
        pltpu.VMEM((K,), jnp.int32),
        pltpu.VMEM((K, D), jnp.float32),
        pltpu.VMEM((8, D), jnp.float32),
        pltpu.VMEM_SHARED((NP, D), jnp.float32),
        pltpu.SemaphoreType.DMA,
    ],
    compiler_params=pltpu.CompilerParams(use_tc_tiling_on_sc=False),
)
def _spmm_partial(x_hbm, g_hbm, s_hbm, v_hbm, out_hbm,
                  gidx_v, sidx_v, vals_v, sidx1_v, rows_v, zero_v, acc_sh, sem):
    c = lax.axis_index("c")
    s = lax.axis_index("s")
    wid = s * NC + c

    ECH = 2000
    def eload(q, _):
        sl = pl.ds(q * ECH, ECH)
        pltpu.sync_copy(g_hbm.at[wid, sl], gidx_v.at[sl])
        pltpu.sync_copy(s_hbm.at[wid, sl], sidx_v.at[sl])
        pltpu.sync_copy(v_hbm.at[wid, sl], vals_v.at[sl])
        return 0
    lax.fori_loop(0, EPT // ECH, eload, 0)

    def zrow(k, _):
        for r in range(D // L):
            zero_v[k, pl.ds(r * L, L)] = jnp.zeros((L,), jnp.float32)
        return 0
    lax.fori_loop(0, 8, zrow, 0)
    def zacc(q, _):
        pltpu.sync_copy(zero_v, acc_sh.at[pl.ds(s * RPT + q * 8, 8)])
        return 0
    lax.fori_loop(0, RPT // 8, zacc, 0)
    plsc.subcore_barrier()

    def chunk(j, _):
        e0 = j * K
        pltpu.async_copy(
            x_hbm.at[gidx_v.at[pl.ds(e0, K)]], rows_v, sem).wait()
        for b in range(NB):
            sidx1_v[pl.ds(b * L, L)] = sidx_v[pl.ds(e0 + b * L, L)]

        def scale16(b, _):
            v16 = vals_v[pl.ds(e0 + b * L, L)]
            for l in range(L):
                bc = _bcast_lane(v16, l)
                k = b * L + l
                for r in range(D // L):
                    sl = pl.ds(r * L, L)
                    rows_v[k, sl] = rows_v[k, sl] * bc
            return 0
        lax.fori_loop(0, NB, scale16, 0)

        pltpu.sync_copy(rows_v, acc_sh.at[sidx1_v], add=True)
        return 0
    lax.fori_loop(0, NSUB, chunk, 0)

    plsc.subcore_barrier()

    for q in range(RPT // RCHUNK):
        off = s * RPT + q * RCHUNK
        pltpu.sync_copy(acc_sh.at[pl.ds(off, RCHUNK)],
                        out_hbm.at[c, pl.ds(off, RCHUNK)])


def _combine(p, leaky):
    def body(p_ref, o_ref):
        x = p_ref[0] + p_ref[1]
        if leaky:
            x = jnp.where(x >= 0, x, LEAKY * x)
        o_ref[...] = x

    rows = 1024
    return pl.pallas_call(
        body,
        out_shape=jax.ShapeDtypeStruct((NP, D), jnp.float32),
        grid=(NP // rows,),
        in_specs=[pl.BlockSpec((2, rows, D), lambda i: (0, i, 0))],
        out_specs=pl.BlockSpec((rows, D), lambda i: (i, 0)),
    )(p)


def kernel(user_emb, item_emb, edge_index, adj_vals):
    x = jnp.concatenate([
        user_emb, item_emb,
        jnp.zeros((NP - N, D), jnp.float32)], axis=0)
    rows = edge_index[0].reshape(NW, EPT)
    cols = edge_index[1].reshape(NW, EPT)
    vals = adj_vals.reshape(NW, EPT)

    for _ in range(2):
        p = _spmm_partial(x, rows, cols, vals)
        t = _combine(p, leaky=False)
        p = _spmm_partial(t, cols, rows, vals)
        x = _combine(p, leaky=True)

    return x[:N_USERS], x[N_USERS:N]
